# baseline (speedup 1.0000x reference)
import functools

import jax
import jax.numpy as jnp
from jax import lax
from jax.experimental import pallas as pl
from jax.experimental.pallas import tpu as pltpu
from jax.experimental.pallas import tpu_sc as plsc

_N = 10000
_E = 320000
_D = 128
_H = 32
_C = 41

_NC = 2
_NS = 16
_NW = _NC * _NS
_B = 128
_CH = 79
_EPAD = _NW * _CH * _B
_NPAD = 10112
_STRIPE = _NPAD // _NS


@functools.cache
def _make_sc_scatter():
    mesh = plsc.VectorSubcoreMesh(
        core_axis_name="c", subcore_axis_name="s",
        num_cores=_NC, num_subcores=_NS)

    @functools.partial(
        pl.kernel,
        out_type=jax.ShapeDtypeStruct((_NC, _NPAD, _H), jnp.float32),
        mesh=mesh,
        compiler_params=pltpu.CompilerParams(use_tc_tiling_on_sc=False),
        scratch_types=[
            pltpu.VMEM((_CH, _B), jnp.int32),
            pltpu.VMEM((_CH, _B), jnp.int32),
            pltpu.VMEM((2, _B, _H), jnp.float32),
            pltpu.VMEM_SHARED((_NPAD, _H), jnp.float32),
            pltpu.SemaphoreType.DMA,
            pltpu.SemaphoreType.DMA,
        ],
    )
    def sc_scatter(u_hbm, src_hbm, dst_hbm, zero_hbm, out_hbm,
                   src_v, dst_v, rows_v, accum, sem0, sem1):
        c = lax.axis_index("c")
        s = lax.axis_index("s")
        wid = s * _NC + c
        pltpu.sync_copy(zero_hbm.at[pl.ds(s * _STRIPE, _STRIPE)],
                        accum.at[pl.ds(s * _STRIPE, _STRIPE)])
        pltpu.sync_copy(src_hbm.at[wid], src_v)
        pltpu.sync_copy(dst_hbm.at[wid], dst_v)
        plsc.subcore_barrier()

        pltpu.async_copy(u_hbm.at[src_v.at[0]], rows_v.at[0], sem0)

        def body2(jj, carry):
            j0 = jj * 2
            j1 = j0 + 1
            pltpu.make_async_copy(u_hbm.at[src_v.at[j0]], rows_v.at[0],
                                  sem0).wait()

            @pl.when(j1 < _CH)
            def _start_j1():
                pltpu.async_copy(u_hbm.at[src_v.at[j1]], rows_v.at[1], sem1)
            pltpu.sync_copy(rows_v.at[0], accum.at[dst_v.at[j0]], add=True)

            @pl.when(j1 < _CH)
            def _do_j1():
                pltpu.make_async_copy(u_hbm.at[src_v.at[j1]], rows_v.at[1],
                                      sem1).wait()

                @pl.when(j1 + 1 < _CH)
                def _start_next():
                    pltpu.async_copy(u_hbm.at[src_v.at[j1 + 1]],
                                     rows_v.at[0], sem0)
                pltpu.sync_copy(rows_v.at[1], accum.at[dst_v.at[j1]],
                                add=True)
            return carry

        lax.fori_loop(0, (_CH + 1) // 2, body2, 0)
        plsc.subcore_barrier()
        pltpu.sync_copy(accum.at[pl.ds(s * _STRIPE, _STRIPE)],
                        out_hbm.at[c].at[pl.ds(s * _STRIPE, _STRIPE)])

    return sc_scatter


def _mm_a_body(x_ref, w_ref, o_ref):
    o_ref[...] = jnp.dot(x_ref[...], w_ref[...],
                         preferred_element_type=jnp.float32,
                precision=lax.Precision.HIGHEST)


def _phase_b_body(u1_ref, s1_ref, b1a_ref, w1b_ref, b1b_ref, g1_ref,
                  bt1_ref, w2a_ref, o_ref):
    p = u1_ref[...] + s1_ref[0] + s1_ref[1] + b1a_ref[...]
    q = jnp.dot(jnp.maximum(p, 0.0), w1b_ref[...],
                preferred_element_type=jnp.float32,
                precision=lax.Precision.HIGHEST) + b1b_ref[...]
    rows = lax.broadcasted_iota(jnp.int32, (_NPAD, _H), 0)
    mask = rows < _N
    qm = jnp.where(mask, q, 0.0)
    mean = jnp.sum(qm, axis=0, keepdims=True) * (1.0 / _N)
    dq = jnp.where(mask, q - mean, 0.0)
    var = jnp.sum(dq * dq, axis=0, keepdims=True) * (1.0 / _N)
    h = g1_ref[...] * (q - mean) * lax.rsqrt(var + 1e-5) + bt1_ref[...]
    u2 = jnp.dot(h, w2a_ref[...], preferred_element_type=jnp.float32,
                precision=lax.Precision.HIGHEST)
    o_ref[...] = jnp.where(mask, u2, 0.0)


def _phase_c_body(u2_ref, s2_ref, b2a_ref, w2b_ref, b2b_ref, g2_ref,
                  bt2_ref, wf1_ref, bf1_ref, wf2_ref, bf2_ref, o_ref):
    p = u2_ref[...] + s2_ref[0] + s2_ref[1] + b2a_ref[...]
    q = jnp.dot(jnp.maximum(p, 0.0), w2b_ref[...],
                preferred_element_type=jnp.float32,
                precision=lax.Precision.HIGHEST) + b2b_ref[...]
    rows = lax.broadcasted_iota(jnp.int32, (_NPAD, _H), 0)
    mask = rows < _N
    qm = jnp.where(mask, q, 0.0)
    mean = jnp.sum(qm, axis=0, keepdims=True) * (1.0 / _N)
    dq = jnp.where(mask, q - mean, 0.0)
    var = jnp.sum(dq * dq, axis=0, keepdims=True) * (1.0 / _N)
    h = g2_ref[...] * (q - mean) * lax.rsqrt(var + 1e-5) + bt2_ref[...]
    f = jnp.maximum(jnp.dot(h, wf1_ref[...],
                            preferred_element_type=jnp.float32,
                precision=lax.Precision.HIGHEST)
                    + bf1_ref[...], 0.0)
    o_ref[...] = jnp.dot(f, wf2_ref[...],
                         preferred_element_type=jnp.float32,
                precision=lax.Precision.HIGHEST) + bf2_ref[...]


_mm_a = pl.pallas_call(
    _mm_a_body,
    out_shape=jax.ShapeDtypeStruct((_NPAD, _H), jnp.float32))

_phase_b = pl.pallas_call(
    _phase_b_body,
    out_shape=jax.ShapeDtypeStruct((_NPAD, _H), jnp.float32))

_phase_c = pl.pallas_call(
    _phase_c_body,
    out_shape=jax.ShapeDtypeStruct((_NPAD, 128), jnp.float32))


def kernel(x, edge_index, W1a, b1a, W1b, b1b, g1, bt1, W2a, b2a, W2b, b2b,
           g2, bt2, Wf1, bf1, Wf2, bf2):
    f32 = jnp.float32
    src = edge_index[0]
    dst = edge_index[1]
    pad_e = _EPAD - _E
    src_p = jnp.concatenate(
        [src, jnp.full((pad_e,), _N, jnp.int32)]).reshape(_NW, _CH, _B)
    dst_p = jnp.concatenate(
        [dst, jnp.full((pad_e,), _N, jnp.int32)]).reshape(_NW, _CH, _B)
    x_p = jnp.zeros((_NPAD, _D), f32).at[:_N].set(x)
    zeros_nh = jnp.zeros((_NPAD, _H), f32)
    wf2_p = jnp.zeros((_H, 128), f32).at[:, :_C].set(Wf2)
    bf2_p = jnp.zeros((128,), f32).at[:_C].set(bf2)

    sc_scatter = _make_sc_scatter()
    u1 = _mm_a(x_p, W1a)
    s1 = sc_scatter(u1, src_p, dst_p, zeros_nh)
    u2 = _phase_b(u1, s1, b1a.reshape(1, _H), W1b, b1b.reshape(1, _H),
                  g1.reshape(1, _H), bt1.reshape(1, _H), W2a)
    s2 = sc_scatter(u2, src_p, dst_p, zeros_nh)
    out = _phase_c(u2, s2, b2a.reshape(1, _H), W2b, b2b.reshape(1, _H),
                   g2.reshape(1, _H), bt2.reshape(1, _H), Wf1,
                   bf1.reshape(1, _H), wf2_p, bf2_p.reshape(1, 128))
    return out[:_N, :_C]


# speedup vs baseline: 1.0101x; 1.0101x over previous
import functools

import jax
import jax.numpy as jnp
from jax import lax
from jax.experimental import pallas as pl
from jax.experimental.pallas import tpu as pltpu
from jax.experimental.pallas import tpu_sc as plsc

_N = 10000
_E = 320000
_D = 128
_H = 32
_C = 41

_NC = 2
_NS = 16
_NW = _NC * _NS
_B = 128
_G = 4
_CH = 80
_EPAD = _NW * _CH * _B
_NPAD = 10112
_STRIPE = _NPAD // _NS


@functools.cache
def _make_sc_scatter():
    mesh = plsc.VectorSubcoreMesh(
        core_axis_name="c", subcore_axis_name="s",
        num_cores=_NC, num_subcores=_NS)

    @functools.partial(
        pl.kernel,
        out_type=jax.ShapeDtypeStruct((_NC, _NPAD, _H), jnp.float32),
        mesh=mesh,
        compiler_params=pltpu.CompilerParams(use_tc_tiling_on_sc=False),
        scratch_types=[
            pltpu.VMEM((_CH, _B), jnp.int32),
            pltpu.VMEM((_CH, _B), jnp.int32),
            pltpu.VMEM((2 * _G, _B, _H), jnp.float32),
            pltpu.VMEM_SHARED((_NPAD, _H), jnp.float32),
            pltpu.SemaphoreType.DMA,
            pltpu.SemaphoreType.DMA,
        ],
    )
    def sc_scatter(u_hbm, src_hbm, dst_hbm, zero_hbm, out_hbm,
                   src_v, dst_v, rows_v, accum, sem0, sem1):
        c = lax.axis_index("c")
        s = lax.axis_index("s")
        wid = s * _NC + c
        pltpu.sync_copy(zero_hbm.at[pl.ds(s * _STRIPE, _STRIPE)],
                        accum.at[pl.ds(s * _STRIPE, _STRIPE)])
        pltpu.sync_copy(src_hbm.at[wid], src_v)
        pltpu.sync_copy(dst_hbm.at[wid], dst_v)
        plsc.subcore_barrier()

        def _gather(j, slot, sem):
            pltpu.async_copy(u_hbm.at[src_v.at[j]], rows_v.at[slot], sem)

        def _drain_group(base, slot0, sem):
            for b in range(_G):
                pltpu.make_async_copy(u_hbm.at[src_v.at[base + b]],
                                      rows_v.at[slot0 + b], sem).wait()
            for b in range(_G):
                pltpu.sync_copy(rows_v.at[slot0 + b],
                                accum.at[dst_v.at[base + b]], add=True)

        for b in range(_G):
            _gather(b, b, sem0)

        n_pairs = _CH // (2 * _G)

        def body(t, carry):
            base0 = (2 * t) * _G
            base1 = base0 + _G
            for b in range(_G):
                _gather(base1 + b, _G + b, sem1)
            _drain_group(base0, 0, sem0)

            @pl.when(t + 1 < n_pairs)
            def _prefetch_next():
                for b in range(_G):
                    _gather(base1 + _G + b, b, sem0)
            _drain_group(base1, _G, sem1)
            return carry

        lax.fori_loop(0, n_pairs, body, 0)
        plsc.subcore_barrier()
        pltpu.sync_copy(accum.at[pl.ds(s * _STRIPE, _STRIPE)],
                        out_hbm.at[c].at[pl.ds(s * _STRIPE, _STRIPE)])

    return sc_scatter


def _mm_a_body(x_ref, w_ref, o_ref):
    o_ref[...] = jnp.dot(x_ref[...], w_ref[...],
                         preferred_element_type=jnp.float32,
                precision=lax.Precision.HIGHEST)


def _phase_b_body(u1_ref, s1_ref, b1a_ref, w1b_ref, b1b_ref, g1_ref,
                  bt1_ref, w2a_ref, o_ref):
    p = u1_ref[...] + s1_ref[0] + s1_ref[1] + b1a_ref[...]
    q = jnp.dot(jnp.maximum(p, 0.0), w1b_ref[...],
                preferred_element_type=jnp.float32,
                precision=lax.Precision.HIGHEST) + b1b_ref[...]
    rows = lax.broadcasted_iota(jnp.int32, (_NPAD, _H), 0)
    mask = rows < _N
    qm = jnp.where(mask, q, 0.0)
    mean = jnp.sum(qm, axis=0, keepdims=True) * (1.0 / _N)
    dq = jnp.where(mask, q - mean, 0.0)
    var = jnp.sum(dq * dq, axis=0, keepdims=True) * (1.0 / _N)
    h = g1_ref[...] * (q - mean) * lax.rsqrt(var + 1e-5) + bt1_ref[...]
    u2 = jnp.dot(h, w2a_ref[...], preferred_element_type=jnp.float32,
                precision=lax.Precision.HIGHEST)
    o_ref[...] = jnp.where(mask, u2, 0.0)


def _phase_c_body(u2_ref, s2_ref, b2a_ref, w2b_ref, b2b_ref, g2_ref,
                  bt2_ref, wf1_ref, bf1_ref, wf2_ref, bf2_ref, o_ref):
    p = u2_ref[...] + s2_ref[0] + s2_ref[1] + b2a_ref[...]
    q = jnp.dot(jnp.maximum(p, 0.0), w2b_ref[...],
                preferred_element_type=jnp.float32,
                precision=lax.Precision.HIGHEST) + b2b_ref[...]
    rows = lax.broadcasted_iota(jnp.int32, (_NPAD, _H), 0)
    mask = rows < _N
    qm = jnp.where(mask, q, 0.0)
    mean = jnp.sum(qm, axis=0, keepdims=True) * (1.0 / _N)
    dq = jnp.where(mask, q - mean, 0.0)
    var = jnp.sum(dq * dq, axis=0, keepdims=True) * (1.0 / _N)
    h = g2_ref[...] * (q - mean) * lax.rsqrt(var + 1e-5) + bt2_ref[...]
    f = jnp.maximum(jnp.dot(h, wf1_ref[...],
                            preferred_element_type=jnp.float32,
                precision=lax.Precision.HIGHEST)
                    + bf1_ref[...], 0.0)
    o_ref[...] = jnp.dot(f, wf2_ref[...],
                         preferred_element_type=jnp.float32,
                precision=lax.Precision.HIGHEST) + bf2_ref[...]


_mm_a = pl.pallas_call(
    _mm_a_body,
    out_shape=jax.ShapeDtypeStruct((_NPAD, _H), jnp.float32))

_phase_b = pl.pallas_call(
    _phase_b_body,
    out_shape=jax.ShapeDtypeStruct((_NPAD, _H), jnp.float32))

_phase_c = pl.pallas_call(
    _phase_c_body,
    out_shape=jax.ShapeDtypeStruct((_NPAD, 128), jnp.float32))


def kernel(x, edge_index, W1a, b1a, W1b, b1b, g1, bt1, W2a, b2a, W2b, b2b,
           g2, bt2, Wf1, bf1, Wf2, bf2):
    f32 = jnp.float32
    src = edge_index[0]
    dst = edge_index[1]
    pad_e = _EPAD - _E
    src_p = jnp.concatenate(
        [src, jnp.full((pad_e,), _N, jnp.int32)]).reshape(_NW, _CH, _B)
    dst_p = jnp.concatenate(
        [dst, jnp.full((pad_e,), _N, jnp.int32)]).reshape(_NW, _CH, _B)
    x_p = jnp.zeros((_NPAD, _D), f32).at[:_N].set(x)
    zeros_nh = jnp.zeros((_NPAD, _H), f32)
    wf2_p = jnp.zeros((_H, 128), f32).at[:, :_C].set(Wf2)
    bf2_p = jnp.zeros((128,), f32).at[:_C].set(bf2)

    sc_scatter = _make_sc_scatter()
    u1 = _mm_a(x_p, W1a)
    s1 = sc_scatter(u1, src_p, dst_p, zeros_nh)
    u2 = _phase_b(u1, s1, b1a.reshape(1, _H), W1b, b1b.reshape(1, _H),
                  g1.reshape(1, _H), bt1.reshape(1, _H), W2a)
    s2 = sc_scatter(u2, src_p, dst_p, zeros_nh)
    out = _phase_c(u2, s2, b2a.reshape(1, _H), W2b, b2b.reshape(1, _H),
                   g2.reshape(1, _H), bt2.reshape(1, _H), Wf1,
                   bf1.reshape(1, _H), wf2_p, bf2_p.reshape(1, 128))
    return out[:_N, :_C]


# speedup vs baseline: 2.2187x; 2.1965x over previous
import functools

import jax
import jax.numpy as jnp
from jax import lax
from jax.experimental import pallas as pl
from jax.experimental.pallas import tpu as pltpu
from jax.experimental.pallas import tpu_sc as plsc

_N = 10000
_E = 320000
_D = 128
_H = 32
_C = 41

_NC = 2
_NS = 16
_NW = _NC * _NS
_B = 128
_G = 4
_CH = 80
_EPAD = _NW * _CH * _B
_NPAD = 10112
_STRIPE = _NPAD // _NS
_NP4 = _NPAD // 4
_NV = _N // 4


@functools.cache
def _make_sc_scatter():
    mesh = plsc.VectorSubcoreMesh(
        core_axis_name="c", subcore_axis_name="s",
        num_cores=_NC, num_subcores=_NS)

    @functools.partial(
        pl.kernel,
        out_type=[jax.ShapeDtypeStruct((_NPAD, _H), jnp.float32),
                  jax.ShapeDtypeStruct((_NPAD, _H), jnp.float32)],
        mesh=mesh,
        compiler_params=pltpu.CompilerParams(use_tc_tiling_on_sc=False),
        scratch_types=[
            pltpu.VMEM((_CH, _B), jnp.int32),
            pltpu.VMEM((_CH, _B), jnp.int32),
            pltpu.VMEM((2 * _G, _B, _H), jnp.float32),
            pltpu.VMEM_SHARED((_NPAD, _H), jnp.float32),
            pltpu.VMEM_SHARED((_NPAD, _H), jnp.float32),
            pltpu.SemaphoreType.DMA,
            pltpu.SemaphoreType.DMA,
        ],
    )
    def sc_scatter(u_hbm, src_hbm, dst_hbm, zero_hbm, out0_hbm, out1_hbm,
                   src_v, dst_v, rows_v, accum, u_sh, sem0, sem1):
        c = lax.axis_index("c")
        s = lax.axis_index("s")
        wid = s * _NC + c
        pltpu.sync_copy(zero_hbm.at[pl.ds(s * _STRIPE, _STRIPE)],
                        accum.at[pl.ds(s * _STRIPE, _STRIPE)])
        pltpu.sync_copy(u_hbm.at[pl.ds(s * _STRIPE, _STRIPE)],
                        u_sh.at[pl.ds(s * _STRIPE, _STRIPE)])
        pltpu.sync_copy(src_hbm.at[wid], src_v)
        pltpu.sync_copy(dst_hbm.at[wid], dst_v)
        plsc.subcore_barrier()

        def _gather(j, slot, sem):
            pltpu.async_copy(u_sh.at[src_v.at[j]], rows_v.at[slot], sem)

        def _drain_group(base, slot0, sem):
            for b in range(_G):
                pltpu.make_async_copy(u_sh.at[src_v.at[base + b]],
                                      rows_v.at[slot0 + b], sem).wait()
            for b in range(_G):
                pltpu.sync_copy(rows_v.at[slot0 + b],
                                accum.at[dst_v.at[base + b]], add=True)

        for b in range(_G):
            _gather(b, b, sem0)

        n_pairs = _CH // (2 * _G)

        def body(t, carry):
            base0 = (2 * t) * _G
            base1 = base0 + _G
            for b in range(_G):
                _gather(base1 + b, _G + b, sem1)
            _drain_group(base0, 0, sem0)

            @pl.when(t + 1 < n_pairs)
            def _prefetch_next():
                for b in range(_G):
                    _gather(base1 + _G + b, b, sem0)
            _drain_group(base1, _G, sem1)
            return carry

        lax.fori_loop(0, n_pairs, body, 0)
        plsc.subcore_barrier()

        @pl.when(c == 0)
        def _pub0():
            pltpu.sync_copy(accum.at[pl.ds(s * _STRIPE, _STRIPE)],
                            out0_hbm.at[pl.ds(s * _STRIPE, _STRIPE)])

        @pl.when(c == 1)
        def _pub1():
            pltpu.sync_copy(accum.at[pl.ds(s * _STRIPE, _STRIPE)],
                            out1_hbm.at[pl.ds(s * _STRIPE, _STRIPE)])

    return sc_scatter


def _mm_a_body(x4_ref, w4_ref, o_ref):
    r = jnp.dot(x4_ref[...], w4_ref[...],
                preferred_element_type=jnp.float32)
    o_ref[...] = jnp.concatenate(
        [r, jnp.zeros((_NP4 - _NV, 128), jnp.float32)], axis=0)


def _bn_packed(q, fold_ref, g_ref, bt_ref):
    rows = lax.broadcasted_iota(jnp.int32, (_NP4, 128), 0)
    mask = rows < _NV
    qm = jnp.where(mask, q, 0.0)
    mean = jnp.dot(jnp.sum(qm, axis=0, keepdims=True), fold_ref[...],
                   preferred_element_type=jnp.float32) * (1.0 / _N)
    dq = jnp.where(mask, q - mean, 0.0)
    var = jnp.dot(jnp.sum(dq * dq, axis=0, keepdims=True), fold_ref[...],
                  preferred_element_type=jnp.float32) * (1.0 / _N)
    h = g_ref[...] * (q - mean) * lax.rsqrt(var + 1e-5) + bt_ref[...]
    return h, mask


def _phase_b_body(u1_ref, s1a_ref, s1b_ref, b1a_ref, w1b_ref, b1b_ref,
                  g1_ref, bt1_ref, w2a_ref, fold_ref, o_ref):
    p = u1_ref[...] + s1a_ref[...] + s1b_ref[...] + b1a_ref[...]
    q = jnp.dot(jnp.maximum(p, 0.0), w1b_ref[...],
                preferred_element_type=jnp.float32) + b1b_ref[...]
    h, mask = _bn_packed(q, fold_ref, g1_ref, bt1_ref)
    u2 = jnp.dot(h, w2a_ref[...], preferred_element_type=jnp.float32)
    o_ref[...] = jnp.where(mask, u2, 0.0)


def _phase_c_body(u2_ref, s2a_ref, s2b_ref, b2a_ref, w2b_ref, b2b_ref,
                  g2_ref, bt2_ref, wf1_ref, bf1_ref, wf2_ref, bf2_ref,
                  fold_ref, o_ref):
    p = u2_ref[...] + s2a_ref[...] + s2b_ref[...] + b2a_ref[...]
    q = jnp.dot(jnp.maximum(p, 0.0), w2b_ref[...],
                preferred_element_type=jnp.float32) + b2b_ref[...]
    h, _ = _bn_packed(q, fold_ref, g2_ref, bt2_ref)
    f = jnp.maximum(jnp.dot(h, wf1_ref[...],
                            preferred_element_type=jnp.float32)
                    + bf1_ref[...], 0.0)
    o_ref[...] = jnp.dot(f, wf2_ref[...],
                         preferred_element_type=jnp.float32) + bf2_ref[...]


_mm_a = pl.pallas_call(
    _mm_a_body,
    out_shape=jax.ShapeDtypeStruct((_NP4, 128), jnp.float32))

_phase_b = pl.pallas_call(
    _phase_b_body,
    out_shape=jax.ShapeDtypeStruct((_NP4, 128), jnp.float32))

_phase_c = pl.pallas_call(
    _phase_c_body,
    out_shape=jax.ShapeDtypeStruct((_NP4, 4 * _C), jnp.float32))


def kernel(x, edge_index, W1a, b1a, W1b, b1b, g1, bt1, W2a, b2a, W2b, b2b,
           g2, bt2, Wf1, bf1, Wf2, bf2):
    f32 = jnp.float32
    src = edge_index[0]
    dst = edge_index[1]
    pad_e = _EPAD - _E
    src_p = jnp.concatenate(
        [src, jnp.full((pad_e,), _N, jnp.int32)]).reshape(_NW, _CH, _B)
    dst_p = jnp.concatenate(
        [dst, jnp.full((pad_e,), _N, jnp.int32)]).reshape(_NW, _CH, _B)
    zeros_nh = jnp.zeros((_NPAD, _H), f32)

    eye4 = jnp.eye(4, dtype=f32)
    fold = jnp.kron(jnp.ones((4, 4), f32), jnp.eye(_H, dtype=f32))

    def t4(v):
        return jnp.tile(v, 4).reshape(1, -1)

    sc_scatter = _make_sc_scatter()
    u1 = _mm_a(x.reshape(_NV, 4 * _D), jnp.kron(eye4, W1a))
    s1a, s1b = sc_scatter(u1.reshape(_NPAD, _H), src_p, dst_p, zeros_nh)
    u2 = _phase_b(u1, s1a.reshape(_NP4, 128), s1b.reshape(_NP4, 128),
                  t4(b1a), jnp.kron(eye4, W1b), t4(b1b), t4(g1), t4(bt1),
                  jnp.kron(eye4, W2a), fold)
    s2a, s2b = sc_scatter(u2.reshape(_NPAD, _H), src_p, dst_p, zeros_nh)
    out = _phase_c(u2, s2a.reshape(_NP4, 128), s2b.reshape(_NP4, 128),
                   t4(b2a), jnp.kron(eye4, W2b), t4(b2b), t4(g2), t4(bt2),
                   jnp.kron(eye4, Wf1), t4(bf1), jnp.kron(eye4, Wf2),
                   t4(bf2), fold)
    return out.reshape(_NPAD, _C)[:_N]
